# baseline (speedup 1.0000x reference)
import functools

import jax
import jax.numpy as jnp
from jax import lax
from jax.experimental import pallas as pl
from jax.experimental.pallas import tpu as pltpu
from jax.experimental.pallas import tpu_sc as plsc

_MAX_LEN = 2048
_HIDDEN = 64
_BATCH = 4096
_HIST = 200

_NC = 2
_NS = 16
_NW = _NC * _NS
_NJT = _HIDDEN // 8
_NHG = _NW // _NJT
_H_PER_G = _HIST // _NHG
_NBT = _BATCH // 128
_LANES = 16

_mesh = plsc.VectorSubcoreMesh(core_axis_name="c", subcore_axis_name="s")


@functools.partial(
    pl.kernel,
    mesh=_mesh,
    out_type=jax.ShapeDtypeStruct((_HIST, _NJT, _NBT, 8, 128), jnp.float32),
    scratch_types=(
        [
            pltpu.VMEM((8, _MAX_LEN), jnp.float32),
            pltpu.VMEM((2, _BATCH), jnp.int16),
            pltpu.VMEM((2, _NBT, 8, 128), jnp.float32),
        ]
        + [pltpu.SemaphoreType.DMA] * 4
    ),
    compiler_params=pltpu.CompilerParams(
        use_tc_tiling_on_sc=False, needs_layout_passes=False
    ),
)
def _gather_kernel(xt_hbm, tabt_hbm, p_hbm, tab_v, idx_v, blk_v, *sems):
    isem = sems[:2]
    wsem = sems[2:]
    wid = lax.axis_index("s") * _NC + lax.axis_index("c")
    jt = wid % _NJT
    hg = wid // _NJT
    h0 = hg * _H_PER_G

    pltpu.sync_copy(tabt_hbm.at[pl.ds(jt * 8, 8)], tab_v)

    jr_splat = [jnp.full((_LANES,), jr, dtype=jnp.int32) for jr in range(8)]

    def idx_load(hi, par):
        return pltpu.make_async_copy(
            xt_hbm.at[h0 + hi], idx_v.at[par], isem[par]
        )

    def blk_write(hi, par):
        return pltpu.make_async_copy(
            blk_v.at[par], p_hbm.at[h0 + hi, jt], wsem[par]
        )

    def compute(par):
        @plsc.parallel_loop(0, _NBT, unroll=2)
        def _(bt):
            for w in range(4):
                iv16 = idx_v[par, pl.ds(bt * 128 + w * 32, 2 * _LANES)]
                iva, ivb = plsc.unpack(
                    iv16,
                    format=plsc.PackFormat.INTERLEAVED,
                    preferred_element_type=jnp.int32,
                )
                for jr in range(8):
                    ga = plsc.load_gather(tab_v, [jr_splat[jr], iva])
                    blk_v[par, bt, jr, pl.ds(w * 32, _LANES)] = ga
                    gb = plsc.load_gather(tab_v, [jr_splat[jr], ivb])
                    blk_v[par, bt, jr, pl.ds(w * 32 + 16, _LANES)] = gb

    idx_load(0, 0).start()

    def body(hh, carry):
        for par in range(2):
            hi = hh * 2 + par

            @pl.when(hi + 1 < _H_PER_G)
            def _():
                idx_load(hi + 1, 1 - par).start()

            idx_load(hi, par).wait()

            @pl.when(hh > 0)
            def _():
                blk_write(hi - 2, par).wait()

            compute(par)
            blk_write(hi, par).start()
        return carry

    lax.fori_loop(0, _H_PER_G // 2, body, 0)

    blk_write(_H_PER_G - 2, 0).wait()
    blk_write(_H_PER_G - 1, 1).wait()


def kernel(x, table):
    xt = x.T.reshape(_HIST, -1, 2, _LANES)
    xt16 = xt.transpose(0, 1, 3, 2).reshape(_HIST, _BATCH).astype(jnp.int16)
    p = _gather_kernel(xt16, table.T)
    return p.transpose(2, 4, 0, 1, 3).reshape(_BATCH, _HIST, _HIDDEN)


# speedup vs baseline: 1.2115x; 1.2115x over previous
import functools

import jax
import jax.numpy as jnp
from jax import lax
from jax.experimental import pallas as pl
from jax.experimental.pallas import tpu as pltpu
from jax.experimental.pallas import tpu_sc as plsc

_MAX_LEN = 2048
_HIDDEN = 64
_BATCH = 4096
_HIST = 200

_NC = 2
_NS = 16
_NW = _NC * _NS
_NJT = _HIDDEN // 8
_NHG = _NW // _NJT
_H_PER_G = _HIST // _NHG
_NBT = _BATCH // 128
_LANES = 16

_mesh = plsc.VectorSubcoreMesh(core_axis_name="c", subcore_axis_name="s")


@functools.partial(
    pl.kernel,
    mesh=_mesh,
    out_type=jax.ShapeDtypeStruct((_HIST, _NJT, _NBT, 8, 128), jnp.float32),
    scratch_types=(
        [
            pltpu.VMEM((8, _MAX_LEN), jnp.float32),
            pltpu.VMEM((2, _BATCH), jnp.int32),
            pltpu.VMEM((2, _NBT, 8, 128), jnp.float32),
        ]
        + [pltpu.SemaphoreType.DMA] * 4
    ),
    compiler_params=pltpu.CompilerParams(
        use_tc_tiling_on_sc=False,
        needs_layout_passes=False,
        disable_bounds_checks=True,
    ),
)
def _gather_kernel(xt_hbm, tabt_hbm, p_hbm, tab_v, idx_v, blk_v, *sems):
    isem = sems[:2]
    wsem = sems[2:]
    wid = lax.axis_index("s") * _NC + lax.axis_index("c")
    jt = wid % _NJT
    hg = wid // _NJT
    h0 = hg * _H_PER_G

    pltpu.sync_copy(tabt_hbm.at[pl.ds(jt * 8, 8)], tab_v)

    jr_splat = [jnp.full((_LANES,), jr, dtype=jnp.int32) for jr in range(8)]

    def idx_load(hi, par):
        return pltpu.make_async_copy(
            xt_hbm.at[h0 + hi], idx_v.at[par], isem[par]
        )

    def blk_write(hi, par):
        return pltpu.make_async_copy(
            blk_v.at[par], p_hbm.at[h0 + hi, jt], wsem[par]
        )

    def compute(par):
        @plsc.parallel_loop(0, _NBT, unroll=2)
        def _(bt):
            for v in range(8):
                iv = idx_v[par, pl.ds(bt * 128 + v * 16, _LANES)]
                for jr in range(8):
                    g = plsc.load_gather(tab_v, [jr_splat[jr], iv])
                    blk_v[par, bt, jr, pl.ds(v * 16, _LANES)] = g

    idx_load(0, 0).start()

    def body(hh, carry):
        for par in range(2):
            hi = hh * 2 + par

            @pl.when(hi + 1 < _H_PER_G)
            def _():
                idx_load(hi + 1, 1 - par).start()

            idx_load(hi, par).wait()

            @pl.when(hh > 0)
            def _():
                blk_write(hi - 2, par).wait()

            compute(par)
            blk_write(hi, par).start()
        return carry

    lax.fori_loop(0, _H_PER_G // 2, body, 0)

    blk_write(_H_PER_G - 2, 0).wait()
    blk_write(_H_PER_G - 1, 1).wait()


def kernel(x, table):
    p = _gather_kernel(x.T, table.T)
    return p.transpose(2, 4, 0, 1, 3).reshape(_BATCH, _HIST, _HIDDEN)


# speedup vs baseline: 1.2125x; 1.0009x over previous
import functools

import jax
import jax.numpy as jnp
from jax import lax
from jax.experimental import pallas as pl
from jax.experimental.pallas import tpu as pltpu
from jax.experimental.pallas import tpu_sc as plsc

_MAX_LEN = 2048
_HIDDEN = 64
_BATCH = 4096
_HIST = 200

_NC = 2
_NS = 16
_NW = _NC * _NS
_NJT = _HIDDEN // 8
_NHG = _NW // _NJT
_H_PER_G = _HIST // _NHG
_NBT = _BATCH // 128
_LANES = 16

_mesh = plsc.VectorSubcoreMesh(core_axis_name="c", subcore_axis_name="s")


@functools.partial(
    pl.kernel,
    mesh=_mesh,
    out_type=jax.ShapeDtypeStruct((_HIST, _NJT, _NBT, 8, 128), jnp.float32),
    scratch_types=(
        [
            pltpu.VMEM((8, _MAX_LEN), jnp.float32),
            pltpu.VMEM((2, _BATCH), jnp.int32),
            pltpu.VMEM((2, _NBT, 8, 128), jnp.float32),
        ]
        + [pltpu.SemaphoreType.DMA] * 4
    ),
    compiler_params=pltpu.CompilerParams(
        use_tc_tiling_on_sc=False, needs_layout_passes=False
    ),
)
def _gather_kernel(xt_hbm, tabt_hbm, p_hbm, tab_v, idx_v, blk_v, *sems):
    isem = sems[:2]
    wsem = sems[2:]
    wid = lax.axis_index("s") * _NC + lax.axis_index("c")
    jt = wid % _NJT
    hg = wid // _NJT
    h0 = hg * _H_PER_G

    pltpu.sync_copy(tabt_hbm.at[pl.ds(jt * 8, 8)], tab_v)

    jr_splat = [jnp.full((_LANES,), jr, dtype=jnp.int32) for jr in range(8)]

    def idx_load(hi, par):
        return pltpu.make_async_copy(
            xt_hbm.at[h0 + hi], idx_v.at[par], isem[par]
        )

    def blk_write(hi, par):
        return pltpu.make_async_copy(
            blk_v.at[par], p_hbm.at[h0 + hi, jt], wsem[par]
        )

    def compute(par):
        @plsc.parallel_loop(0, _NBT, unroll=2)
        def _(bt):
            for v in range(8):
                iv = idx_v[par, pl.ds(bt * 128 + v * 16, _LANES)]
                for jr in range(8):
                    g = plsc.load_gather(tab_v, [jr_splat[jr], iv])
                    blk_v[par, bt, jr, pl.ds(v * 16, _LANES)] = g

    idx_load(0, 0).start()

    def body(hh, carry):
        for par in range(2):
            hi = hh * 2 + par

            @pl.when(hi + 1 < _H_PER_G)
            def _():
                idx_load(hi + 1, 1 - par).start()

            idx_load(hi, par).wait()

            @pl.when(hh > 0)
            def _():
                blk_write(hi - 2, par).wait()

            compute(par)
            blk_write(hi, par).start()
        return carry

    lax.fori_loop(0, _H_PER_G // 2, body, 0)

    blk_write(_H_PER_G - 2, 0).wait()
    blk_write(_H_PER_G - 1, 1).wait()


def kernel(x, table):
    p = _gather_kernel(x.T, table.T)
    return p.transpose(2, 4, 0, 1, 3).reshape(_BATCH, _HIST, _HIDDEN)


# speedup vs baseline: 1.4900x; 1.2288x over previous
import functools

import jax
import jax.numpy as jnp
from jax import lax
from jax.experimental import pallas as pl
from jax.experimental.pallas import tpu as pltpu
from jax.experimental.pallas import tpu_sc as plsc

_MAX_LEN = 2048
_HIDDEN = 64
_BATCH = 4096
_HIST = 200

_NC = 2
_NS = 16
_NW = _NC * _NS
_NJT = _HIDDEN // 8
_NHG = _NW // _NJT
_H_PER_G = _HIST // _NHG
_NBT = _BATCH // 128
_LANES = 16

_mesh = plsc.VectorSubcoreMesh(core_axis_name="c", subcore_axis_name="s")


@functools.partial(
    pl.kernel,
    mesh=_mesh,
    out_type=jax.ShapeDtypeStruct((_HIST, _NJT, _NBT, 8, 128), jnp.float32),
    scratch_types=(
        [
            pltpu.VMEM((8, _MAX_LEN), jnp.float32),
            pltpu.VMEM((2, _BATCH), jnp.int32),
            pltpu.VMEM((2, _NBT, 8, 128), jnp.float32),
        ]
        + [pltpu.SemaphoreType.DMA] * 4
    ),
    compiler_params=pltpu.CompilerParams(
        use_tc_tiling_on_sc=False, needs_layout_passes=False
    ),
)
def _gather_kernel(xt_hbm, tabt_hbm, p_hbm, tab_v, idx_v, blk_v, *sems):
    isem = sems[:2]
    wsem = sems[2:]
    wid = lax.axis_index("s") * _NC + lax.axis_index("c")
    jt = wid % _NJT
    hg = wid // _NJT
    h0 = hg * _H_PER_G

    pltpu.sync_copy(tabt_hbm.at[pl.ds(jt * 8, 8)], tab_v)

    jr_splat = [jnp.full((_LANES,), jr, dtype=jnp.int32) for jr in range(8)]

    def idx_load(hi, par):
        return pltpu.make_async_copy(
            xt_hbm.at[h0 + hi], idx_v.at[par], isem[par]
        )

    def blk_write(hi, par):
        return pltpu.make_async_copy(
            blk_v.at[par], p_hbm.at[h0 + hi, jt], wsem[par]
        )

    def compute(par):
        @plsc.parallel_loop(0, _NBT * 8, unroll=4)
        def _(it):
            bt = it // 8
            v = it % 8
            iv = idx_v[par, pl.ds(it * 16, _LANES)]
            for jr in range(8):
                g = plsc.load_gather(tab_v, [jr_splat[jr], iv])
                blk_v[par, bt, jr, pl.ds(v * 16, _LANES)] = g

    idx_load(0, 0).start()

    def body(hh, carry):
        for par in range(2):
            hi = hh * 2 + par

            @pl.when(hi + 1 < _H_PER_G)
            def _():
                idx_load(hi + 1, 1 - par).start()

            idx_load(hi, par).wait()

            @pl.when(hh > 0)
            def _():
                blk_write(hi - 2, par).wait()

            compute(par)
            blk_write(hi, par).start()
        return carry

    lax.fori_loop(0, _H_PER_G // 2, body, 0)

    blk_write(_H_PER_G - 2, 0).wait()
    blk_write(_H_PER_G - 1, 1).wait()


def kernel(x, table):
    p = _gather_kernel(x.T, table.T)
    return p.transpose(2, 4, 0, 1, 3).reshape(_BATCH, _HIST, _HIDDEN)
